# Initial kernel scaffold; baseline (speedup 1.0000x reference)
import jax
import jax.numpy as jnp
from jax.experimental import pallas as pl


def kernel(node_ids, edge_index, edge_attr, batch, W1_0, b1_0, W2_0, b2_0, root0, bias0, W1_1, b1_1, W2_1, b2_1, root1, bias1, C1, cb1, C2, cb2):
    raise NotImplementedError("write your pallas kernel here")


# speedup vs baseline: 3.1038x; 3.1038x over previous
import functools

import jax
import jax.numpy as jnp
from jax import lax
from jax.experimental import pallas as pl
from jax.experimental.pallas import tpu as pltpu
from jax.experimental.pallas import tpu_sc as plsc

N = 10000
E = 320000
G = 64
L = 16

NC, NS = 2, 16
NW = NC * NS
EPW = E // NW
CH = 80
NCHUNK = EPW // CH
NPAD = 10240
RPT = NPAD // NS

_mesh = plsc.VectorSubcoreMesh(
    core_axis_name="c", subcore_axis_name="s", num_cores=NC, num_subcores=NS)


def _worker(base_name="c"):
  cid = lax.axis_index("c")
  sid = lax.axis_index("s")
  return sid * NC + cid, cid, sid


def _zero_rows(zero_v, shared, sid):
  def zfill(i, _):
    zero_v[i, :] = jnp.zeros((L,), jnp.float32)
    return 0
  lax.fori_loop(0, RPT, zfill, 0, unroll=8)
  for sh in shared:
    pltpu.sync_copy(zero_v, sh.at[pl.ds(sid * RPT, RPT)])


@functools.partial(
    pl.kernel,
    out_type=(jax.ShapeDtypeStruct((NC, NPAD, L), jnp.float32),
              jax.ShapeDtypeStruct((NC, NPAD, L), jnp.float32)),
    mesh=_mesh,
    compiler_params=pltpu.CompilerParams(use_tc_tiling_on_sc=False),
    scratch_types=[
        pltpu.VMEM((CH,), jnp.int32),
        pltpu.VMEM((CH,), jnp.int32),
        pltpu.VMEM((CH, L), jnp.float32),
        pltpu.VMEM((CH, L), jnp.float32),
        pltpu.VMEM((CH, L), jnp.float32),
        pltpu.VMEM((CH, L), jnp.float32),
        pltpu.VMEM((RPT, L), jnp.float32),
        pltpu.VMEM_SHARED((NPAD, L), jnp.float32),
        pltpu.VMEM_SHARED((NPAD, L), jnp.float32),
        pltpu.SemaphoreType.DMA,
    ])
def _sc_msg0(x0_hbm, h0_hbm, src_hbm, dst_hbm, agg_out, cnt_out,
             src_v, dst_v, xs_v, h0_v, msg_v, ones_v, zero_v,
             agg_sh, cnt_sh, sem):
  wid, cid, sid = _worker()

  def ofill(i, _):
    ones_v[i, :] = jnp.ones((L,), jnp.float32)
    return 0
  lax.fori_loop(0, CH, ofill, 0, unroll=8)
  _zero_rows(zero_v, (agg_sh, cnt_sh), sid)
  plsc.subcore_barrier()

  base0 = wid * EPW

  def chunk(i, _):
    base = base0 + i * CH
    pltpu.sync_copy(src_hbm.at[pl.ds(base, CH)], src_v)
    pltpu.sync_copy(dst_hbm.at[pl.ds(base, CH)], dst_v)
    pltpu.sync_copy(h0_hbm.at[pl.ds(base, CH)], h0_v)
    pltpu.async_copy(x0_hbm.at[src_v], xs_v, sem).wait()

    def mul(e, _):
      msg_v[e, :] = xs_v[e, :] * h0_v[e, :]
      return 0
    lax.fori_loop(0, CH, mul, 0, unroll=8)

    pltpu.sync_copy(msg_v, agg_sh.at[dst_v], add=True)
    pltpu.sync_copy(ones_v, cnt_sh.at[dst_v], add=True)
    return 0

  lax.fori_loop(0, NCHUNK, chunk, 0)
  plsc.subcore_barrier()
  pltpu.sync_copy(agg_sh.at[pl.ds(sid * RPT, RPT)],
                  agg_out.at[cid, pl.ds(sid * RPT, RPT)])
  pltpu.sync_copy(cnt_sh.at[pl.ds(sid * RPT, RPT)],
                  cnt_out.at[cid, pl.ds(sid * RPT, RPT)])


@functools.partial(
    pl.kernel,
    out_type=jax.ShapeDtypeStruct((E, L), jnp.float32),
    mesh=_mesh,
    compiler_params=pltpu.CompilerParams(use_tc_tiling_on_sc=False),
    scratch_types=[
        pltpu.VMEM((CH,), jnp.int32),
        pltpu.VMEM((CH, L), jnp.float32),
        pltpu.SemaphoreType.DMA,
    ])
def _sc_gather(x1_hbm, src_hbm, out_hbm, idx_v, rows_v, sem):
  wid, cid, sid = _worker()
  base0 = wid * EPW

  def chunk(i, _):
    base = base0 + i * CH
    pltpu.sync_copy(src_hbm.at[pl.ds(base, CH)], idx_v)
    pltpu.async_copy(x1_hbm.at[idx_v], rows_v, sem).wait()
    pltpu.sync_copy(rows_v, out_hbm.at[pl.ds(base, CH)])
    return 0

  lax.fori_loop(0, NCHUNK, chunk, 0)


@functools.partial(
    pl.kernel,
    out_type=jax.ShapeDtypeStruct((NC, NPAD, L), jnp.float32),
    mesh=_mesh,
    compiler_params=pltpu.CompilerParams(use_tc_tiling_on_sc=False),
    scratch_types=[
        pltpu.VMEM((CH,), jnp.int32),
        pltpu.VMEM((CH, L), jnp.float32),
        pltpu.VMEM((RPT, L), jnp.float32),
        pltpu.VMEM_SHARED((NPAD, L), jnp.float32),
        pltpu.SemaphoreType.DMA,
    ])
def _sc_scatter(msg_hbm, dst_hbm, agg_out, dst_v, msg_v, zero_v, agg_sh, sem):
  wid, cid, sid = _worker()
  _zero_rows(zero_v, (agg_sh,), sid)
  plsc.subcore_barrier()

  base0 = wid * EPW

  def chunk(i, _):
    base = base0 + i * CH
    pltpu.sync_copy(dst_hbm.at[pl.ds(base, CH)], dst_v)
    pltpu.sync_copy(msg_hbm.at[pl.ds(base, CH)], msg_v)
    pltpu.sync_copy(msg_v, agg_sh.at[dst_v], add=True)
    return 0

  lax.fori_loop(0, NCHUNK, chunk, 0)
  plsc.subcore_barrier()
  pltpu.sync_copy(agg_sh.at[pl.ds(sid * RPT, RPT)],
                  agg_out.at[cid, pl.ds(sid * RPT, RPT)])


def _h0_body(ea, w1, b1, w2, b2, out):
  r = jnp.maximum(jnp.dot(ea[...], w1[...]) + b1[...], 0.0)
  out[...] = jnp.dot(r, w2[...]) + b2[...]


def _h0_call(ea, w1, b1, w2, b2):
  BE = 3200
  full = lambda i: (0, 0)
  return pl.pallas_call(
      _h0_body,
      grid=(E // BE,),
      in_specs=[
          pl.BlockSpec((BE, L), lambda i: (i, 0)),
          pl.BlockSpec((L, L), full),
          pl.BlockSpec((1, L), full),
          pl.BlockSpec((L, L), full),
          pl.BlockSpec((1, L), full),
      ],
      out_specs=pl.BlockSpec((BE, L), lambda i: (i, 0)),
      out_shape=jax.ShapeDtypeStruct((E, L), jnp.float32),
  )(ea, w1, b1, w2, b2)


def _msg1_body(ea, xs, w1, b1, w2, b2, rmat, smat, out):
  r = jnp.maximum(jnp.dot(ea[...], w1[...]) + b1[...], 0.0)
  h = jnp.dot(r, w2[...]) + b2[...]
  xe = jnp.dot(xs[...], rmat[...])
  out[...] = jnp.dot(h * xe, smat[...])


def _msg1_call(ea, xs, w1, b1, w2, b2, rmat, smat):
  BE = 1280
  full = lambda i: (0, 0)
  return pl.pallas_call(
      _msg1_body,
      grid=(E // BE,),
      in_specs=[
          pl.BlockSpec((BE, L), lambda i: (i, 0)),
          pl.BlockSpec((BE, L), lambda i: (i, 0)),
          pl.BlockSpec((L, 256), full),
          pl.BlockSpec((1, 256), full),
          pl.BlockSpec((256, 256), full),
          pl.BlockSpec((1, 256), full),
          pl.BlockSpec((L, 256), full),
          pl.BlockSpec((256, L), full),
      ],
      out_specs=pl.BlockSpec((BE, L), lambda i: (i, 0)),
      out_shape=jax.ShapeDtypeStruct((E, L), jnp.float32),
  )(ea, xs, w1, b1, w2, b2, rmat, smat)


def _node0_body(a0, a1, c0, c1, x, root0, bias0, x1_out, invc_out):
  c = c0[:N] + c1[:N]
  invc = 1.0 / jnp.maximum(c, 1.0)
  agg = (a0[:N] + a1[:N]) * invc
  x1_out[...] = jnp.maximum(agg + x[...] * root0[...] + bias0[...], 0.0)
  invc_out[...] = invc


def _node0_call(a0, a1, c0, c1, x, root0, bias0):
  return pl.pallas_call(
      _node0_body,
      out_shape=(jax.ShapeDtypeStruct((N, L), jnp.float32),
                 jax.ShapeDtypeStruct((N, L), jnp.float32)),
  )(a0, a1, c0, c1, x, root0, bias0)


def _final_body(a0, a1, invc, x1, root1, bias1, batch, c1, cb1, c2, cb2, out):
  agg = (a0[:N] + a1[:N]) * invc[...]
  x2 = jnp.maximum(agg + jnp.dot(x1[...], root1[...]) + bias1[...], 0.0)
  gids = lax.broadcasted_iota(jnp.int32, (G, N), 0)
  onehot = (batch[...] == gids).astype(jnp.float32)
  psum = jnp.dot(onehot, x2)
  gcnt = jnp.sum(onehot, axis=1, keepdims=True)
  pooled = psum / jnp.maximum(gcnt, 1.0)
  h = jnp.maximum(jnp.dot(pooled, c1[...]) + cb1[...], 0.0)
  out[...] = jnp.dot(h, c2[...]) + cb2[...]


def _final_call(a0, a1, invc, x1, root1, bias1, batch2d, c1, cb1, c2, cb2):
  return pl.pallas_call(
      _final_body,
      out_shape=jax.ShapeDtypeStruct((G, 4), jnp.float32),
  )(a0, a1, invc, x1, root1, bias1, batch2d, c1, cb1, c2, cb2)


def kernel(node_ids, edge_index, edge_attr, batch,
           W1_0, b1_0, W2_0, b2_0, root0, bias0,
           W1_1, b1_1, W2_1, b2_1, root1, bias1,
           C1, cb1, C2, cb2):
  src = edge_index[0]
  dst = edge_index[1]

  x0 = jnp.broadcast_to(node_ids, (N, L))

  h0 = _h0_call(edge_attr, W1_0, b1_0.reshape(1, L), W2_0, b2_0.reshape(1, L))
  agg0p, cnt0p = _sc_msg0(x0, h0, src, dst)
  x1, invc = _node0_call(agg0p[0], agg0p[1], cnt0p[0], cnt0p[1],
                         node_ids, root0, bias0.reshape(1, L))

  xs1 = _sc_gather(x1, src)
  rmat = jnp.repeat(jnp.eye(L, dtype=jnp.float32), L, axis=1)
  smat = jnp.tile(jnp.eye(L, dtype=jnp.float32), (L, 1))
  msg1 = _msg1_call(edge_attr, xs1, W1_1, b1_1.reshape(1, 256),
                    W2_1, b2_1.reshape(1, 256), rmat, smat)
  agg1p = _sc_scatter(msg1, dst)

  out = _final_call(agg1p[0], agg1p[1], invc, x1,
                    root1, bias1.reshape(1, L), batch.reshape(1, N),
                    C1, cb1.reshape(1, L), C2, cb2.reshape(1, 4))
  return out


# speedup vs baseline: 6.4143x; 2.0666x over previous
import functools

import jax
import jax.numpy as jnp
from jax import lax
from jax.experimental import pallas as pl
from jax.experimental.pallas import tpu as pltpu
from jax.experimental.pallas import tpu_sc as plsc

N = 10000
E = 320000
G = 64
L = 16

NC, NS = 2, 16
NW = NC * NS
EPW = E // NW
CH = 80
NCHUNK = EPW // CH
NPAD = 10240
RPT = NPAD // NS
HS = 2000
SLABCH = HS // CH

_mesh = plsc.VectorSubcoreMesh(
    core_axis_name="c", subcore_axis_name="s", num_cores=NC, num_subcores=NS)


def _worker(base_name="c"):
  cid = lax.axis_index("c")
  sid = lax.axis_index("s")
  return sid * NC + cid, cid, sid


def _zero_rows(zero_v, shared, sid):
  def zfill(i, _):
    zero_v[i, :] = jnp.zeros((L,), jnp.float32)
    return 0
  lax.fori_loop(0, RPT, zfill, 0, unroll=8)
  for sh in shared:
    pltpu.sync_copy(zero_v, sh.at[pl.ds(sid * RPT, RPT)])


@functools.partial(
    pl.kernel,
    out_type=jax.ShapeDtypeStruct((NC, NPAD, L), jnp.float32),
    mesh=_mesh,
    compiler_params=pltpu.CompilerParams(use_tc_tiling_on_sc=False),
    scratch_types=[
        pltpu.VMEM((EPW,), jnp.int32),
        pltpu.VMEM((CH,), jnp.int32),
        pltpu.VMEM((CH, L), jnp.float32),
        pltpu.VMEM((RPT, L), jnp.float32),
        pltpu.VMEM_SHARED((NPAD, L), jnp.float32),
    ])
def _sc_counts(dst_hbm, cnt_out, dst_all, dst_v, ones_v, zero_v, cnt_sh):
  wid, cid, sid = _worker()

  def ofill(i, _):
    ones_v[i, :] = jnp.ones((L,), jnp.float32)
    return 0
  lax.fori_loop(0, CH, ofill, 0, unroll=8)
  _zero_rows(zero_v, (cnt_sh,), sid)
  base0 = wid * EPW
  pltpu.sync_copy(dst_hbm.at[pl.ds(base0, EPW)], dst_all)
  plsc.subcore_barrier()

  def chunk(c, _):
    for t in range(CH // L):
      dst_v[pl.ds(t * L, L)] = dst_all[pl.ds(c * CH + t * L, L)]
    pltpu.sync_copy(ones_v, cnt_sh.at[dst_v], add=True)
    return 0

  lax.fori_loop(0, NCHUNK, chunk, 0)
  plsc.subcore_barrier()
  pltpu.sync_copy(cnt_sh.at[pl.ds(sid * RPT, RPT)],
                  cnt_out.at[cid, pl.ds(sid * RPT, RPT)])


@functools.partial(
    pl.kernel,
    out_type=jax.ShapeDtypeStruct((NC, NPAD, L), jnp.float32),
    mesh=_mesh,
    compiler_params=pltpu.CompilerParams(use_tc_tiling_on_sc=False),
    scratch_types=[
        pltpu.VMEM((EPW,), jnp.int32),
        pltpu.VMEM((CH,), jnp.int32),
        pltpu.VMEM((HS, L), jnp.float32),
        pltpu.VMEM((HS // 8, 128), jnp.float32),
        pltpu.VMEM((CH, L), jnp.float32),
        pltpu.VMEM((RPT, L), jnp.float32),
        pltpu.VMEM_SHARED((NPAD, L), jnp.float32),
        pltpu.SemaphoreType.DMA,
    ])
def _sc_msg0(xs_hbm, h0_hbm, dst_hbm, agg_out,
             dst_all, dst_v, xs_slab, h0_slab, msg_v, zero_v, agg_sh, sem):
  wid, cid, sid = _worker()
  _zero_rows(zero_v, (agg_sh,), sid)
  base0 = wid * EPW
  pltpu.sync_copy(dst_hbm.at[pl.ds(base0, EPW)], dst_all)
  plsc.subcore_barrier()

  def chunk(c, _):
    @pl.when(lax.rem(c, SLABCH) == 0)
    def _():
      pltpu.sync_copy(xs_hbm.at[pl.ds(base0 + c * CH, HS)], xs_slab)
      pltpu.sync_copy(
          h0_hbm.at[pl.ds((base0 + c * CH) // 8, HS // 8)], h0_slab)
    j = lax.rem(c, SLABCH)
    jrow = j * (CH // 8)
    jxs = j * CH
    for r in range(CH // 8):
      for cc in range(8):
        msg_v[8 * r + cc, :] = (
            xs_slab[jxs + 8 * r + cc, :]
            * h0_slab[jrow + r, cc * L:(cc + 1) * L])
    for t in range(CH // L):
      dst_v[pl.ds(t * L, L)] = dst_all[pl.ds(c * CH + t * L, L)]
    pltpu.sync_copy(msg_v, agg_sh.at[dst_v], add=True)
    return 0

  lax.fori_loop(0, NCHUNK, chunk, 0)
  plsc.subcore_barrier()
  pltpu.sync_copy(agg_sh.at[pl.ds(sid * RPT, RPT)],
                  agg_out.at[cid, pl.ds(sid * RPT, RPT)])


@functools.partial(
    pl.kernel,
    out_type=jax.ShapeDtypeStruct((E, L), jnp.float32),
    mesh=_mesh,
    compiler_params=pltpu.CompilerParams(use_tc_tiling_on_sc=False),
    scratch_types=[
        pltpu.VMEM((EPW,), jnp.int32),
        pltpu.VMEM((CH,), jnp.int32),
        pltpu.VMEM((CH,), jnp.int32),
        pltpu.VMEM((CH,), jnp.int32),
        pltpu.VMEM((CH,), jnp.int32),
        pltpu.VMEM((CH, L), jnp.float32),
        pltpu.VMEM((CH, L), jnp.float32),
        pltpu.VMEM((CH, L), jnp.float32),
        pltpu.VMEM((CH, L), jnp.float32),
        pltpu.SemaphoreType.DMA,
        pltpu.SemaphoreType.DMA,
    ])
def _sc_gather(x1_hbm, src_hbm, out_hbm, src_all,
               i0, i1, i2, i3, r0, r1, r2, r3, sem_g, sem_w):
  wid, cid, sid = _worker()
  base0 = wid * EPW
  idxs = (i0, i1, i2, i3)
  rows = (r0, r1, r2, r3)
  pltpu.sync_copy(src_hbm.at[pl.ds(base0, EPW)], src_all)

  def idx_chunk(dst_ref, c):
    for t in range(CH // L):
      dst_ref[pl.ds(t * L, L)] = src_all[pl.ds(c * CH + t * L, L)]

  def fire(c, b):
    idx_chunk(idxs[b], c)
    pltpu.async_copy(x1_hbm.at[idxs[b]], rows[b], sem_g)

  def wait_g(b):
    pltpu.make_async_copy(x1_hbm.at[pl.ds(0, CH)], rows[b], sem_g).wait()

  def wait_w(b):
    pltpu.make_async_copy(rows[b], out_hbm.at[pl.ds(0, CH)], sem_w).wait()

  fire(0, 0)
  fire(1, 1)

  def quad(k, _):
    for b in range(4):
      s = 4 * k + b

      @pl.when(s >= 2)
      def _():
        wait_w(b)

      @pl.when(s + 2 < NCHUNK)
      def _():
        fire(s + 2, (b + 2) % 4)
      wait_g(b)
      pltpu.async_copy(rows[b], out_hbm.at[pl.ds(base0 + s * CH, CH)], sem_w)
    return 0

  lax.fori_loop(0, NCHUNK // 4, quad, 0)
  sT = NCHUNK - 1
  wait_g(0)
  pltpu.async_copy(rows[0], out_hbm.at[pl.ds(base0 + sT * CH, CH)], sem_w)
  wait_w(2)
  wait_w(3)
  wait_w(0)


@functools.partial(
    pl.kernel,
    out_type=jax.ShapeDtypeStruct((NC, NPAD, L), jnp.float32),
    mesh=_mesh,
    compiler_params=pltpu.CompilerParams(use_tc_tiling_on_sc=False),
    scratch_types=[
        pltpu.VMEM((EPW,), jnp.int32),
        pltpu.VMEM((CH,), jnp.int32),
        pltpu.VMEM((HS, L), jnp.float32),
        pltpu.VMEM((RPT, L), jnp.float32),
        pltpu.VMEM_SHARED((NPAD, L), jnp.float32),
        pltpu.SemaphoreType.DMA,
    ])
def _sc_scatter(msg_hbm, dst_hbm, agg_out, dst_all, dst_v, msg_slab, zero_v,
                agg_sh, sem):
  wid, cid, sid = _worker()
  _zero_rows(zero_v, (agg_sh,), sid)

  base0 = wid * EPW
  pltpu.sync_copy(dst_hbm.at[pl.ds(base0, EPW)], dst_all)
  plsc.subcore_barrier()

  def chunk(c, _):
    @pl.when(lax.rem(c, SLABCH) == 0)
    def _():
      pltpu.sync_copy(msg_hbm.at[pl.ds(base0 + c * CH, HS)], msg_slab)
    for t in range(CH // L):
      dst_v[pl.ds(t * L, L)] = dst_all[pl.ds(c * CH + t * L, L)]
    pltpu.sync_copy(msg_slab.at[pl.ds(lax.rem(c, SLABCH) * CH, CH)],
                    agg_sh.at[dst_v], add=True)
    return 0

  lax.fori_loop(0, NCHUNK, chunk, 0)
  plsc.subcore_barrier()
  pltpu.sync_copy(agg_sh.at[pl.ds(sid * RPT, RPT)],
                  agg_out.at[cid, pl.ds(sid * RPT, RPT)])


BE = 2560
BR = BE // 8
NB = E // BE


def _pack(x):
  return jnp.concatenate([x[c * BR:(c + 1) * BR, :] for c in range(8)],
                         axis=1)


def _unpack(xp):
  return jnp.concatenate([xp[:, c * L:(c + 1) * L] for c in range(8)],
                         axis=0)


def _h0_body(ea, w1d, b1t, w2d, b2t, out):
  bf = jnp.bfloat16
  f32 = jnp.float32
  eap = _pack(ea[...])
  r = jnp.maximum(
      jnp.dot(eap.astype(bf), w1d[...].astype(bf),
              preferred_element_type=f32) + b1t[...], 0.0)
  out[...] = jnp.dot(r.astype(bf), w2d[...].astype(bf),
                     preferred_element_type=f32) + b2t[...]


def _h0_call(ea, w1d, b1t, w2d, b2t):
  full = lambda i: (0, 0)
  return pl.pallas_call(
      _h0_body,
      grid=(NB,),
      in_specs=[
          pl.BlockSpec((BE, L), lambda i: (i, 0)),
          pl.BlockSpec((128, 128), full),
          pl.BlockSpec((1, 128), full),
          pl.BlockSpec((128, 128), full),
          pl.BlockSpec((1, 128), full),
      ],
      out_specs=pl.BlockSpec((BR, 128), lambda i: (i, 0)),
      out_shape=jax.ShapeDtypeStruct((E // 8, 128), jnp.float32),
  )(ea, w1d, b1t, w2d, b2t)


def _msg1_body(ea, xsp, w1, b1, w2, b2, rmat, smat, out):
  bf = jnp.bfloat16
  f32 = jnp.float32
  r = jnp.maximum(
      jnp.dot(ea[...].astype(bf), w1[...].astype(bf),
              preferred_element_type=f32) + b1[...], 0.0)
  h = jnp.dot(r.astype(bf), w2[...].astype(bf),
              preferred_element_type=f32) + b2[...]
  xs = _unpack(xsp[...])
  xe = jnp.dot(xs.astype(bf), rmat[...].astype(bf),
               preferred_element_type=f32)
  msg = jnp.dot((h * xe).astype(bf), smat[...].astype(bf),
                preferred_element_type=f32)
  out[...] = _pack(msg)


def _msg1_call(ea, xsp, w1, b1, w2, b2, rmat, smat):
  full = lambda i: (0, 0)
  return pl.pallas_call(
      _msg1_body,
      grid=(NB,),
      in_specs=[
          pl.BlockSpec((BE, L), lambda i: (i, 0)),
          pl.BlockSpec((BR, 128), lambda i: (i, 0)),
          pl.BlockSpec((L, 256), full),
          pl.BlockSpec((1, 256), full),
          pl.BlockSpec((256, 256), full),
          pl.BlockSpec((1, 256), full),
          pl.BlockSpec((L, 256), full),
          pl.BlockSpec((256, L), full),
      ],
      out_specs=pl.BlockSpec((BR, 128), lambda i: (i, 0)),
      out_shape=jax.ShapeDtypeStruct((E // 8, 128), jnp.float32),
  )(ea, xsp, w1, b1, w2, b2, rmat, smat)


def _node0_body(a0, a1, c0, c1, x, root0, bias0, x1_out, invc_out):
  c = c0[:N] + c1[:N]
  invc = 1.0 / jnp.maximum(c, 1.0)
  agg = (a0[:N] + a1[:N]) * invc
  x1_out[...] = jnp.maximum(agg + x[...] * root0[...] + bias0[...], 0.0)
  invc_out[...] = invc


def _node0_call(a0, a1, c0, c1, x, root0, bias0):
  return pl.pallas_call(
      _node0_body,
      out_shape=(jax.ShapeDtypeStruct((N, L), jnp.float32),
                 jax.ShapeDtypeStruct((N, L), jnp.float32)),
  )(a0, a1, c0, c1, x, root0, bias0)


def _final_body(a0, a1, invc, x1, root1, bias1, batch, c1, cb1, c2, cb2, out):
  agg = (a0[:N] + a1[:N]) * invc[...]
  x2 = jnp.maximum(agg + jnp.dot(x1[...], root1[...]) + bias1[...], 0.0)
  gids = lax.broadcasted_iota(jnp.int32, (G, N), 0)
  onehot = (batch[...] == gids).astype(jnp.float32)
  psum = jnp.dot(onehot, x2)
  gcnt = jnp.sum(onehot, axis=1, keepdims=True)
  pooled = psum / jnp.maximum(gcnt, 1.0)
  h = jnp.maximum(jnp.dot(pooled, c1[...]) + cb1[...], 0.0)
  out[...] = jnp.dot(h, c2[...]) + cb2[...]


def _final_call(a0, a1, invc, x1, root1, bias1, batch2d, c1, cb1, c2, cb2):
  return pl.pallas_call(
      _final_body,
      out_shape=jax.ShapeDtypeStruct((G, 4), jnp.float32),
  )(a0, a1, invc, x1, root1, bias1, batch2d, c1, cb1, c2, cb2)


def kernel(node_ids, edge_index, edge_attr, batch,
           W1_0, b1_0, W2_0, b2_0, root0, bias0,
           W1_1, b1_1, W2_1, b2_1, root1, bias1,
           C1, cb1, C2, cb2):
  srcp = edge_index[0].reshape(NB, 8, BR).transpose(0, 2, 1).reshape(E)
  dstp = edge_index[1].reshape(NB, 8, BR).transpose(0, 2, 1).reshape(E)

  x0 = jnp.broadcast_to(node_ids, (N, L))

  cnt0p = _sc_counts(dstp)
  xs0 = _sc_gather(x0, srcp)
  eye8 = jnp.eye(8, dtype=jnp.float32)
  h0 = _h0_call(edge_attr, jnp.kron(eye8, W1_0), jnp.tile(b1_0, 8).reshape(1, 128),
                jnp.kron(eye8, W2_0), jnp.tile(b2_0, 8).reshape(1, 128))
  agg0p = _sc_msg0(xs0, h0, dstp)
  x1, invc = _node0_call(agg0p[0], agg0p[1], cnt0p[0], cnt0p[1],
                         node_ids, root0, bias0.reshape(1, L))

  xsp = _sc_gather(x1, srcp).reshape(E // 8, 128)
  rmat = jnp.repeat(jnp.eye(L, dtype=jnp.float32), L, axis=1)
  smat = jnp.tile(jnp.eye(L, dtype=jnp.float32), (L, 1))
  msg1 = _msg1_call(edge_attr, xsp, W1_1, b1_1.reshape(1, 256),
                    W2_1, b2_1.reshape(1, 256), rmat, smat)
  agg1p = _sc_scatter(msg1.reshape(E, L), dstp)

  out = _final_call(agg1p[0], agg1p[1], invc, x1,
                    root1, bias1.reshape(1, L), batch.reshape(1, N),
                    C1, cb1.reshape(1, L), C2, cb2.reshape(1, 4))
  return out


# speedup vs baseline: 6.5056x; 1.0142x over previous
import functools

import jax
import jax.numpy as jnp
from jax import lax
from jax.experimental import pallas as pl
from jax.experimental.pallas import tpu as pltpu
from jax.experimental.pallas import tpu_sc as plsc

N = 10000
E = 320000
G = 64
L = 16

NC, NS = 2, 16
NW = NC * NS
EPW = E // NW
CH = 80
NCHUNK = EPW // CH
NPAD = 10240
RPT = NPAD // NS
HS = 2000
SLABCH = HS // CH

_mesh = plsc.VectorSubcoreMesh(
    core_axis_name="c", subcore_axis_name="s", num_cores=NC, num_subcores=NS)


def _worker(base_name="c"):
  cid = lax.axis_index("c")
  sid = lax.axis_index("s")
  return sid * NC + cid, cid, sid


def _zero_rows(zero_v, shared, sid):
  def zfill(i, _):
    zero_v[i, :] = jnp.zeros((L,), jnp.float32)
    return 0
  lax.fori_loop(0, RPT, zfill, 0, unroll=8)
  for sh in shared:
    pltpu.sync_copy(zero_v, sh.at[pl.ds(sid * RPT, RPT)])


@functools.partial(
    pl.kernel,
    out_type=jax.ShapeDtypeStruct((NC, NPAD, L), jnp.float32),
    mesh=_mesh,
    compiler_params=pltpu.CompilerParams(use_tc_tiling_on_sc=False),
    scratch_types=[
        pltpu.VMEM((EPW,), jnp.int32),
        pltpu.VMEM((CH,), jnp.int32),
        pltpu.VMEM((CH, L), jnp.float32),
        pltpu.VMEM((RPT, L), jnp.float32),
        pltpu.VMEM_SHARED((NPAD, L), jnp.float32),
    ])
def _sc_counts(dst_hbm, cnt_out, dst_all, dst_v, ones_v, zero_v, cnt_sh):
  wid, cid, sid = _worker()

  def ofill(i, _):
    ones_v[i, :] = jnp.ones((L,), jnp.float32)
    return 0
  lax.fori_loop(0, CH, ofill, 0, unroll=8)
  _zero_rows(zero_v, (cnt_sh,), sid)
  base0 = wid * EPW
  pltpu.sync_copy(dst_hbm.at[pl.ds(base0, EPW)], dst_all)
  plsc.subcore_barrier()

  def chunk(c, _):
    for t in range(CH // L):
      dst_v[pl.ds(t * L, L)] = dst_all[pl.ds(c * CH + t * L, L)]
    pltpu.sync_copy(ones_v, cnt_sh.at[dst_v], add=True)
    return 0

  lax.fori_loop(0, NCHUNK, chunk, 0)
  plsc.subcore_barrier()
  pltpu.sync_copy(cnt_sh.at[pl.ds(sid * RPT, RPT)],
                  cnt_out.at[cid, pl.ds(sid * RPT, RPT)])


@functools.partial(
    pl.kernel,
    out_type=jax.ShapeDtypeStruct((NC, NPAD, L), jnp.float32),
    mesh=_mesh,
    compiler_params=pltpu.CompilerParams(use_tc_tiling_on_sc=False),
    scratch_types=[
        pltpu.VMEM((EPW,), jnp.int32),
        pltpu.VMEM((CH,), jnp.int32),
        pltpu.VMEM((HS, L), jnp.float32),
        pltpu.VMEM((HS // 8, 128), jnp.float32),
        pltpu.VMEM((CH, L), jnp.float32),
        pltpu.VMEM((RPT, L), jnp.float32),
        pltpu.VMEM_SHARED((NPAD, L), jnp.float32),
        pltpu.SemaphoreType.DMA,
    ])
def _sc_msg0(xs_hbm, h0_hbm, dst_hbm, agg_out,
             dst_all, dst_v, xs_slab, h0_slab, msg_v, zero_v, agg_sh, sem):
  wid, cid, sid = _worker()
  _zero_rows(zero_v, (agg_sh,), sid)
  base0 = wid * EPW
  pltpu.sync_copy(dst_hbm.at[pl.ds(base0, EPW)], dst_all)
  plsc.subcore_barrier()

  def chunk(c, _):
    @pl.when(lax.rem(c, SLABCH) == 0)
    def _():
      pltpu.sync_copy(xs_hbm.at[pl.ds(base0 + c * CH, HS)], xs_slab)
      pltpu.sync_copy(
          h0_hbm.at[pl.ds((base0 + c * CH) // 8, HS // 8)], h0_slab)
    j = lax.rem(c, SLABCH)
    jrow = j * (CH // 8)
    jxs = j * CH
    for r in range(CH // 8):
      for cc in range(8):
        msg_v[8 * r + cc, :] = (
            xs_slab[jxs + 8 * r + cc, :]
            * h0_slab[jrow + r, cc * L:(cc + 1) * L])
    for t in range(CH // L):
      dst_v[pl.ds(t * L, L)] = dst_all[pl.ds(c * CH + t * L, L)]
    pltpu.sync_copy(msg_v, agg_sh.at[dst_v], add=True)
    return 0

  lax.fori_loop(0, NCHUNK, chunk, 0)
  plsc.subcore_barrier()
  pltpu.sync_copy(agg_sh.at[pl.ds(sid * RPT, RPT)],
                  agg_out.at[cid, pl.ds(sid * RPT, RPT)])


@functools.partial(
    pl.kernel,
    out_type=jax.ShapeDtypeStruct((E, L), jnp.float32),
    mesh=_mesh,
    compiler_params=pltpu.CompilerParams(use_tc_tiling_on_sc=False),
    scratch_types=[
        pltpu.VMEM((EPW,), jnp.int32),
        pltpu.VMEM((CH,), jnp.int32),
        pltpu.VMEM((CH,), jnp.int32),
        pltpu.VMEM((CH,), jnp.int32),
        pltpu.VMEM((CH,), jnp.int32),
        pltpu.VMEM((CH, L), jnp.float32),
        pltpu.VMEM((CH, L), jnp.float32),
        pltpu.VMEM((CH, L), jnp.float32),
        pltpu.VMEM((CH, L), jnp.float32),
        pltpu.SemaphoreType.DMA,
        pltpu.SemaphoreType.DMA,
    ])
def _sc_gather(x1_hbm, src_hbm, out_hbm, src_all,
               i0, i1, i2, i3, r0, r1, r2, r3, sem_g, sem_w):
  wid, cid, sid = _worker()
  base0 = wid * EPW
  idxs = (i0, i1, i2, i3)
  rows = (r0, r1, r2, r3)
  pltpu.sync_copy(src_hbm.at[pl.ds(base0, EPW)], src_all)

  def idx_chunk(dst_ref, c):
    for t in range(CH // L):
      dst_ref[pl.ds(t * L, L)] = src_all[pl.ds(c * CH + t * L, L)]

  def fire(c, b):
    idx_chunk(idxs[b], c)
    pltpu.async_copy(x1_hbm.at[idxs[b]], rows[b], sem_g)

  def wait_g(b):
    pltpu.make_async_copy(x1_hbm.at[pl.ds(0, CH)], rows[b], sem_g).wait()

  def wait_w(b):
    pltpu.make_async_copy(rows[b], out_hbm.at[pl.ds(0, CH)], sem_w).wait()

  fire(0, 0)
  fire(1, 1)

  def quad(k, _):
    for b in range(4):
      s = 4 * k + b

      @pl.when(s >= 2)
      def _():
        wait_w(b)

      @pl.when(s + 2 < NCHUNK)
      def _():
        fire(s + 2, (b + 2) % 4)
      wait_g(b)
      pltpu.async_copy(rows[b], out_hbm.at[pl.ds(base0 + s * CH, CH)], sem_w)
    return 0

  lax.fori_loop(0, NCHUNK // 4, quad, 0)
  sT = NCHUNK - 1
  wait_g(0)
  pltpu.async_copy(rows[0], out_hbm.at[pl.ds(base0 + sT * CH, CH)], sem_w)
  wait_w(2)
  wait_w(3)
  wait_w(0)


@functools.partial(
    pl.kernel,
    out_type=jax.ShapeDtypeStruct((NC, NPAD, L), jnp.float32),
    mesh=_mesh,
    compiler_params=pltpu.CompilerParams(use_tc_tiling_on_sc=False),
    scratch_types=[
        pltpu.VMEM((EPW,), jnp.int32),
        pltpu.VMEM((CH,), jnp.int32),
        pltpu.VMEM((HS, L), jnp.float32),
        pltpu.VMEM((RPT, L), jnp.float32),
        pltpu.VMEM_SHARED((NPAD, L), jnp.float32),
        pltpu.SemaphoreType.DMA,
    ])
def _sc_scatter(msg_hbm, dst_hbm, agg_out, dst_all, dst_v, msg_slab, zero_v,
                agg_sh, sem):
  wid, cid, sid = _worker()
  _zero_rows(zero_v, (agg_sh,), sid)

  base0 = wid * EPW
  pltpu.sync_copy(dst_hbm.at[pl.ds(base0, EPW)], dst_all)
  plsc.subcore_barrier()

  def chunk(c, _):
    @pl.when(lax.rem(c, SLABCH) == 0)
    def _():
      pltpu.sync_copy(msg_hbm.at[pl.ds(base0 + c * CH, HS)], msg_slab)
    for t in range(CH // L):
      dst_v[pl.ds(t * L, L)] = dst_all[pl.ds(c * CH + t * L, L)]
    pltpu.sync_copy(msg_slab.at[pl.ds(lax.rem(c, SLABCH) * CH, CH)],
                    agg_sh.at[dst_v], add=True)
    return 0

  lax.fori_loop(0, NCHUNK, chunk, 0)
  plsc.subcore_barrier()
  pltpu.sync_copy(agg_sh.at[pl.ds(sid * RPT, RPT)],
                  agg_out.at[cid, pl.ds(sid * RPT, RPT)])


BE = 2560
BR = BE // 8
NB = E // BE


def _pack(x):
  return jnp.concatenate([x[c * BR:(c + 1) * BR, :] for c in range(8)],
                         axis=1)


def _unpack(xp):
  return jnp.concatenate([xp[:, c * L:(c + 1) * L] for c in range(8)],
                         axis=0)


def _h0_body(ea, w1d, b1t, w2d, b2t, out):
  bf = jnp.bfloat16
  f32 = jnp.float32
  eap = _pack(ea[...])
  r = jnp.maximum(
      jnp.dot(eap.astype(bf), w1d[...].astype(bf),
              preferred_element_type=f32) + b1t[...], 0.0)
  out[...] = jnp.dot(r.astype(bf), w2d[...].astype(bf),
                     preferred_element_type=f32) + b2t[...]


def _h0_call(ea, w1d, b1t, w2d, b2t):
  full = lambda i: (0, 0)
  return pl.pallas_call(
      _h0_body,
      grid=(NB,),
      in_specs=[
          pl.BlockSpec((BE, L), lambda i: (i, 0)),
          pl.BlockSpec((128, 128), full),
          pl.BlockSpec((1, 128), full),
          pl.BlockSpec((128, 128), full),
          pl.BlockSpec((1, 128), full),
      ],
      out_specs=pl.BlockSpec((BR, 128), lambda i: (i, 0)),
      out_shape=jax.ShapeDtypeStruct((E // 8, 128), jnp.float32),
  )(ea, w1d, b1t, w2d, b2t)


def _msg1_body(ea, xsp, w1, b1, w2, b2, rmat, smat, out):
  bf = jnp.bfloat16
  f32 = jnp.float32
  r = jnp.maximum(
      jnp.dot(ea[...].astype(bf), w1[...].astype(bf),
              preferred_element_type=f32) + b1[...], 0.0)
  h = jnp.dot(r.astype(bf), w2[...].astype(bf),
              preferred_element_type=f32) + b2[...]
  xs = _unpack(xsp[...])
  xe = jnp.dot(xs.astype(bf), rmat[...].astype(bf),
               preferred_element_type=f32)
  msg = jnp.dot((h * xe).astype(bf), smat[...].astype(bf),
                preferred_element_type=f32)
  out[...] = _pack(msg)


def _msg1_call(ea, xsp, w1, b1, w2, b2, rmat, smat):
  full = lambda i: (0, 0)
  return pl.pallas_call(
      _msg1_body,
      grid=(NB,),
      in_specs=[
          pl.BlockSpec((BE, L), lambda i: (i, 0)),
          pl.BlockSpec((BR, 128), lambda i: (i, 0)),
          pl.BlockSpec((L, 256), full),
          pl.BlockSpec((1, 256), full),
          pl.BlockSpec((256, 256), full),
          pl.BlockSpec((1, 256), full),
          pl.BlockSpec((L, 256), full),
          pl.BlockSpec((256, L), full),
      ],
      out_specs=pl.BlockSpec((BR, 128), lambda i: (i, 0)),
      out_shape=jax.ShapeDtypeStruct((E // 8, 128), jnp.float32),
  )(ea, xsp, w1, b1, w2, b2, rmat, smat)


NR = N // 8


def _node0_body(a0, a1, c0, c1, x0p, root0t, bias0t, x1tab_out, x1p_out,
                invc_out):
  c = c0[:NR] + c1[:NR]
  invc = 1.0 / jnp.maximum(c, 1.0)
  agg = (a0[:NR] + a1[:NR]) * invc
  x1p = jnp.maximum(agg + x0p[...] * root0t[...] + bias0t[...], 0.0)
  x1tab_out[...] = jnp.concatenate(
      [x1p[:, c * L:(c + 1) * L] for c in range(8)], axis=0)
  x1p_out[...] = x1p
  invc_out[...] = invc


def _node0_call(a0, a1, c0, c1, x0p, root0t, bias0t):
  return pl.pallas_call(
      _node0_body,
      out_shape=(jax.ShapeDtypeStruct((N, L), jnp.float32),
                 jax.ShapeDtypeStruct((NR, 128), jnp.float32),
                 jax.ShapeDtypeStruct((NR, 128), jnp.float32)),
  )(a0, a1, c0, c1, x0p, root0t, bias0t)


def _final_body(a0, a1, invc, x1p, root1d, bias1t, batchp, c1, cb1, c2, cb2,
                out):
  agg = (a0[:NR] + a1[:NR]) * invc[...]
  x2p = jnp.maximum(
      agg + jnp.dot(x1p[...], root1d[...]) + bias1t[...], 0.0)
  x2 = jnp.concatenate(
      [x2p[:, c * L:(c + 1) * L] for c in range(8)], axis=0)
  gids = lax.broadcasted_iota(jnp.int32, (G, N), 0)
  onehot = (batchp[...] == gids).astype(jnp.float32)
  psum = jnp.dot(onehot, x2)
  gcnt = jnp.sum(onehot, axis=1, keepdims=True)
  pooled = psum / jnp.maximum(gcnt, 1.0)
  h = jnp.maximum(jnp.dot(pooled, c1[...]) + cb1[...], 0.0)
  out[...] = jnp.dot(h, c2[...]) + cb2[...]


def _final_call(a0, a1, invc, x1p, root1d, bias1t, batchp, c1, cb1, c2, cb2):
  return pl.pallas_call(
      _final_body,
      out_shape=jax.ShapeDtypeStruct((G, 4), jnp.float32),
  )(a0, a1, invc, x1p, root1d, bias1t, batchp, c1, cb1, c2, cb2)


def kernel(node_ids, edge_index, edge_attr, batch,
           W1_0, b1_0, W2_0, b2_0, root0, bias0,
           W1_1, b1_1, W2_1, b2_1, root1, bias1,
           C1, cb1, C2, cb2):
  srcp = edge_index[0].reshape(NB, 8, BR).transpose(0, 2, 1).reshape(E)
  dstp = edge_index[1].reshape(NB, 8, BR).transpose(0, 2, 1).reshape(E)

  x0 = jnp.broadcast_to(node_ids, (N, L))

  cnt0p = _sc_counts(dstp)
  xs0 = _sc_gather(x0, srcp)
  eye8 = jnp.eye(8, dtype=jnp.float32)
  h0 = _h0_call(edge_attr, jnp.kron(eye8, W1_0), jnp.tile(b1_0, 8).reshape(1, 128),
                jnp.kron(eye8, W2_0), jnp.tile(b2_0, 8).reshape(1, 128))
  agg0p = _sc_msg0(xs0, h0, dstp)

  pk = lambda a: a.reshape(NPAD // 8, 128)
  x0p = jnp.repeat(node_ids.reshape(NR, 8), L, axis=1)
  x1tab, x1p, invcp = _node0_call(
      pk(agg0p[0]), pk(agg0p[1]), pk(cnt0p[0]), pk(cnt0p[1]),
      x0p, jnp.tile(root0.reshape(L), 8).reshape(1, 128),
      jnp.tile(bias0, 8).reshape(1, 128))

  srcg = lax.rem(srcp, 8) * NR + srcp // 8
  xsp = _sc_gather(x1tab, srcg).reshape(E // 8, 128)
  rmat = jnp.repeat(jnp.eye(L, dtype=jnp.float32), L, axis=1)
  smat = jnp.tile(jnp.eye(L, dtype=jnp.float32), (L, 1))
  msg1 = _msg1_call(edge_attr, xsp, W1_1, b1_1.reshape(1, 256),
                    W2_1, b2_1.reshape(1, 256), rmat, smat)
  agg1p = _sc_scatter(msg1.reshape(E, L), dstp)

  batchp = batch.reshape(NR, 8).transpose().reshape(1, N)
  out = _final_call(
      pk(agg1p[0]), pk(agg1p[1]), invcp, x1p,
      jnp.kron(jnp.eye(8, dtype=jnp.float32), root1),
      jnp.tile(bias1, 8).reshape(1, 128), batchp,
      C1, cb1.reshape(1, L), C2, cb2.reshape(1, 4))
  return out
